# baseline (speedup 1.0000x reference)
import functools

import jax
import jax.numpy as jnp
from jax import lax
from jax.experimental import pallas as pl
from jax.experimental.pallas import tpu as pltpu
from jax.experimental.pallas import tpu_sc as plsc

N = 10000
E = 320000
D = 128
H = 64
C = 40

LANES = 128
NC = 2
NS = 16
NW = NC * NS
ROWS_PW = 80
ROWS_PAD = ROWS_PW * NW
EPAD = ROWS_PAD * LANES
SR = 40
STAGES_PW = 2 * ROWS_PW // SR
SLOW_STAGES = 2
SLOW_C = 0
NPAD = 10240

_sc_mesh = plsc.VectorSubcoreMesh(core_axis_name="c", subcore_axis_name="s")
_sc_params = pltpu.CompilerParams(use_tc_tiling_on_sc=False,
                                 needs_layout_passes=False)


@functools.partial(
    pl.kernel,
    out_type=jax.ShapeDtypeStruct((NC, NPAD), jnp.float32),
    mesh=_sc_mesh,
    scratch_types=[
        pltpu.VMEM((ROWS_PW, LANES), jnp.int32),
        pltpu.VMEM((ROWS_PW, LANES), jnp.float32),
        pltpu.VMEM_SHARED((NPAD,), jnp.float32),
        pltpu.SemaphoreType.DMA,
        pltpu.SemaphoreType.DMA,
    ],
    compiler_params=_sc_params,
)
def _deg_kernel(dst_hbm, w_hbm, z_hbm, out_hbm, dst_v, w_v, deg_sh, sem0, sem1):
    c = lax.axis_index("c")
    s = lax.axis_index("s")
    chunk = NPAD // NS
    pltpu.sync_copy(z_hbm.at[pl.ds(s * chunk, chunk)],
                    deg_sh.at[pl.ds(s * chunk, chunk)])

    wid = s * NC + c
    base = wid * ROWS_PW
    cp0 = pltpu.async_copy(dst_hbm.at[pl.ds(base, ROWS_PW)], dst_v, sem0)
    cp1 = pltpu.async_copy(w_hbm.at[pl.ds(base, ROWS_PW)], w_v, sem1)
    cp0.wait()
    cp1.wait()
    plsc.subcore_barrier()

    def body(r, _):
        pltpu.sync_copy(w_v.at[r], deg_sh.at[dst_v.at[r]], add=True)
        return 0

    lax.fori_loop(0, ROWS_PW, body, 0)
    plsc.subcore_barrier()
    pltpu.sync_copy(deg_sh.at[pl.ds(s * chunk, chunk)],
                    out_hbm.at[c, pl.ds(s * chunk, chunk)])


@functools.partial(
    pl.kernel,
    out_type=jax.ShapeDtypeStruct((NC, NPAD, H), jnp.float32),
    mesh=_sc_mesh,
    scratch_types=[
        pltpu.VMEM((SR, LANES), jnp.int32),
        pltpu.VMEM((SR, LANES), jnp.int32),
        pltpu.VMEM((SR, LANES), jnp.float32),
        pltpu.VMEM((4, LANES, H), jnp.bfloat16),
        pltpu.VMEM((4, LANES, H), jnp.float32),
        pltpu.VMEM_SHARED((NPAD, H), jnp.float32),
        pltpu.VMEM_SHARED((N, H), jnp.bfloat16),
        pltpu.SemaphoreType.DMA,
        pltpu.SemaphoreType.DMA,
        pltpu.SemaphoreType.DMA,
        pltpu.SemaphoreType.DMA,
        pltpu.SemaphoreType.DMA,
        pltpu.SemaphoreType.DMA,
        pltpu.SemaphoreType.DMA,
        pltpu.SemaphoreType.DMA,
    ],
    compiler_params=_sc_params,
)
def _edge_kernel(xs_hbm, src_hbm, dst_hbm, w_hbm, z_hbm, out_hbm,
                 src_v, dst_v, w_v, gbuf, sbuf, acc_sh, xs_sh,
                 g0, g1, g2, g3, s0, s1, s2, s3):
    c = lax.axis_index("c")
    s = lax.axis_index("s")
    nrows = NPAD // NS
    gsem = (g0, g1, g2, g3)
    ssem = (s0, s1, s2, s3)

    cpz = pltpu.async_copy(z_hbm.at[pl.ds(s * nrows, nrows)],
                           acc_sh.at[pl.ds(s * nrows, nrows)], s1)
    pltpu.sync_copy(xs_hbm.at[pl.ds(s * (N // NS), N // NS)],
                    xs_sh.at[pl.ds(s * (N // NS), N // NS)])
    cpz.wait()

    slow = c == SLOW_C
    nstages = jnp.where(slow, SLOW_STAGES, STAGES_PW - SLOW_STAGES)
    base = jnp.where(
        slow,
        s * SLOW_STAGES * SR,
        NS * SLOW_STAGES * SR + s * (STAGES_PW - SLOW_STAGES) * SR)
    plsc.subcore_barrier()

    def gather(r, b):
        pltpu.async_copy(xs_sh.at[src_v.at[r]], gbuf.at[b], gsem[b])

    def wait_gather(r, b):
        pltpu.make_async_copy(xs_sh.at[src_v.at[r]], gbuf.at[b],
                              gsem[b]).wait()

    def scale(r, b):
        def grp(g, _):
            wv = w_v[r, pl.ds(g * 16, 16)]
            for j in range(16):
                e = g * 16 + j
                we = wv[j]
                a0, a1 = plsc.unpack(gbuf[b, e, pl.ds(0, 32)],
                                     format=plsc.PackFormat.INTERLEAVED)
                a2, a3 = plsc.unpack(gbuf[b, e, pl.ds(32, 32)],
                                     format=plsc.PackFormat.INTERLEAVED)
                sbuf[b, e, pl.ds(0, 16)] = a0 * we
                sbuf[b, e, pl.ds(16, 16)] = a1 * we
                sbuf[b, e, pl.ds(32, 16)] = a2 * we
                sbuf[b, e, pl.ds(48, 16)] = a3 * we
            return 0

        lax.fori_loop(0, LANES // 16, grp, 0)

    def scatter(r, b):
        pltpu.async_copy(sbuf.at[b], acc_sh.at[dst_v.at[r]], ssem[b],
                         add=True)

    def wait_scatter(r, b):
        pltpu.make_async_copy(sbuf.at[b], acc_sh.at[dst_v.at[r]],
                              ssem[b]).wait()

    def stage(h, _):
        hb = base + h * SR
        cp0 = pltpu.async_copy(src_hbm.at[pl.ds(hb, SR)], src_v, g0)
        cp1 = pltpu.async_copy(dst_hbm.at[pl.ds(hb, SR)], dst_v, g1)
        cp2 = pltpu.async_copy(w_hbm.at[pl.ds(hb, SR)], w_v, s0)
        cp0.wait()
        cp1.wait()
        cp2.wait()

        for b in range(4):
            gather(b, b)

        def body(i, _):
            for b in range(4):
                r = 4 * i + b
                wait_gather(r, b)

                @pl.when(i > 0)
                def _():
                    wait_scatter(r - 4, b)

                scale(r, b)
                scatter(r, b)

                @pl.when(i < SR // 4 - 1)
                def _():
                    gather(r + 4, b)

            return 0

        lax.fori_loop(0, SR // 4, body, 0)
        for b in range(4):
            wait_scatter(SR - 4 + b, b)
        return 0

    lax.fori_loop(0, nstages, stage, 0)
    plsc.subcore_barrier()
    pltpu.sync_copy(acc_sh.at[pl.ds(s * nrows, nrows)],
                    out_hbm.at[c, pl.ds(s * nrows, nrows)])


_RB = 1000
_GRID = (N // _RB,)


def _tc1_body(deg0_ref, deg1_ref, x_ref, w1_ref, dinv_ref, xs_ref):
    deg = 1.0 + deg0_ref[...] + deg1_ref[...]
    dinv = jnp.where(deg > 0, lax.rsqrt(jnp.maximum(deg, 1e-12)), 0.0)
    dinv_ref[...] = dinv
    xw = jnp.dot(x_ref[...], w1_ref[...], preferred_element_type=jnp.float32)
    xs_ref[...] = xw * dinv


def _tc1(deg0, deg1, x, W1):
    return pl.pallas_call(
        _tc1_body,
        grid=_GRID,
        in_specs=[
            pl.BlockSpec((_RB, 1), lambda i: (i, 0)),
            pl.BlockSpec((_RB, 1), lambda i: (i, 0)),
            pl.BlockSpec((_RB, D), lambda i: (i, 0)),
            pl.BlockSpec((D, H), lambda i: (0, 0)),
        ],
        out_specs=[
            pl.BlockSpec((_RB, 1), lambda i: (i, 0)),
            pl.BlockSpec((_RB, H), lambda i: (i, 0)),
        ],
        out_shape=[
            jax.ShapeDtypeStruct((N, 1), jnp.float32),
            jax.ShapeDtypeStruct((N, H), jnp.float32),
        ],
    )(deg0, deg1, x, W1)


def _tc2_body(p_ref, q_ref, xs_ref, dinv_ref, b_ref, w_ref, out_ref):
    dinv = dinv_ref[...]
    h = dinv * (p_ref[0] + q_ref[0] + xs_ref[...]) + b_ref[...]
    h = jnp.maximum(h, 0.0)
    xw = jnp.dot(h, w_ref[...], preferred_element_type=jnp.float32)
    out_ref[...] = xw * dinv


def _tc2(p, xs, dinv, b1, W2):
    return pl.pallas_call(
        _tc2_body,
        grid=_GRID,
        in_specs=[
            pl.BlockSpec((1, _RB, H), lambda i: (0, i, 0)),
            pl.BlockSpec((1, _RB, H), lambda i: (1, i, 0)),
            pl.BlockSpec((_RB, H), lambda i: (i, 0)),
            pl.BlockSpec((_RB, 1), lambda i: (i, 0)),
            pl.BlockSpec((1, H), lambda i: (0, 0)),
            pl.BlockSpec((H, H), lambda i: (0, 0)),
        ],
        out_specs=pl.BlockSpec((_RB, H), lambda i: (i, 0)),
        out_shape=jax.ShapeDtypeStruct((N, H), jnp.float32),
    )(p, p, xs, dinv, b1, W2)


def _tc3_body(q0_ref, q1_ref, xs_ref, dinv_ref, b2_ref, wm1_ref, bm1_ref,
              wm2_ref, bm2_ref, out_ref):
    h = dinv_ref[...] * (q0_ref[0] + q1_ref[0] + xs_ref[...]) + b2_ref[...]
    h = jnp.maximum(h, 0.0)
    h = jnp.dot(h, wm1_ref[...], preferred_element_type=jnp.float32) + bm1_ref[...]
    h = jnp.maximum(h, 0.0)
    logits = jnp.dot(h, wm2_ref[...], preferred_element_type=jnp.float32) + bm2_ref[...]
    m = jnp.max(logits, axis=-1, keepdims=True)
    e = jnp.exp(logits - m)
    out_ref[...] = e / jnp.sum(e, axis=-1, keepdims=True)


def _tc3(q0, q1, xs, dinv, b2, Wm1, bm1, Wm2, bm2):
    return pl.pallas_call(
        _tc3_body,
        grid=_GRID,
        in_specs=[
            pl.BlockSpec((1, _RB, H), lambda i: (0, i, 0)),
            pl.BlockSpec((1, _RB, H), lambda i: (1, i, 0)),
            pl.BlockSpec((_RB, H), lambda i: (i, 0)),
            pl.BlockSpec((_RB, 1), lambda i: (i, 0)),
            pl.BlockSpec((1, H), lambda i: (0, 0)),
            pl.BlockSpec((H, H), lambda i: (0, 0)),
            pl.BlockSpec((1, H), lambda i: (0, 0)),
            pl.BlockSpec((H, C), lambda i: (0, 0)),
            pl.BlockSpec((1, C), lambda i: (0, 0)),
        ],
        out_specs=pl.BlockSpec((_RB, C), lambda i: (i, 0)),
        out_shape=jax.ShapeDtypeStruct((N, C), jnp.float32),
    )(q0, q1, xs, dinv, b2, Wm1, bm1, Wm2, bm2)


def _ileave(xs):
    t = xs.reshape(N, 2, 2, 16).transpose(0, 1, 3, 2)
    return t.reshape(N, H).astype(jnp.bfloat16)


def kernel(x, edge_index, edge_attr, W1, b1, W2, b2, Wm1, bm1, Wm2, bm2):
    pad = EPAD - E
    src = jnp.concatenate(
        [edge_index[0], jnp.zeros((pad,), jnp.int32)]).reshape(ROWS_PAD, LANES)
    dst = jnp.concatenate(
        [edge_index[1], jnp.zeros((pad,), jnp.int32)]).reshape(ROWS_PAD, LANES)
    w = jnp.concatenate(
        [edge_attr, jnp.zeros((pad,), jnp.float32)]).reshape(ROWS_PAD, LANES)

    zcol = jnp.zeros((NPAD,), jnp.float32)
    zacc = jnp.zeros((NPAD, H), jnp.float32)
    degp = _deg_kernel(dst, w, zcol)
    deg0 = degp[0, :N].reshape(N, 1)
    deg1 = degp[1, :N].reshape(N, 1)

    dinv, xs1 = _tc1(deg0, deg1, x, W1)

    p = _edge_kernel(_ileave(xs1), src, dst, w, zacc)
    xs2 = _tc2(p, xs1, dinv, b1.reshape(1, H), W2)

    q = _edge_kernel(_ileave(xs2), src, dst, w, zacc)
    return _tc3(q, q, xs2, dinv, b2.reshape(1, H),
                Wm1, bm1.reshape(1, H), Wm2, bm2.reshape(1, C))


# speedup vs baseline: 1.0903x; 1.0903x over previous
import functools

import jax
import jax.numpy as jnp
from jax import lax
from jax.experimental import pallas as pl
from jax.experimental.pallas import tpu as pltpu
from jax.experimental.pallas import tpu_sc as plsc

N = 10000
E = 320000
D = 128
H = 64
C = 40

LANES = 128
NC = 2
NS = 16
NW = NC * NS
ROWS_PW = 80
ROWS_PAD = ROWS_PW * NW
EPAD = ROWS_PAD * LANES
SR = 40
STAGES_PW = 2 * ROWS_PW // SR
SLOW_STAGES = 2
SLOW_C = 0
NPAD = 10240

_sc_mesh = plsc.VectorSubcoreMesh(core_axis_name="c", subcore_axis_name="s")
_sc_params = pltpu.CompilerParams(use_tc_tiling_on_sc=False)


@functools.partial(
    pl.kernel,
    out_type=jax.ShapeDtypeStruct((NC, NPAD), jnp.float32),
    mesh=_sc_mesh,
    scratch_types=[
        pltpu.VMEM((ROWS_PW, LANES), jnp.int32),
        pltpu.VMEM((ROWS_PW, LANES), jnp.float32),
        pltpu.VMEM_SHARED((NPAD,), jnp.float32),
        pltpu.SemaphoreType.DMA,
        pltpu.SemaphoreType.DMA,
    ],
    compiler_params=_sc_params,
)
def _deg_kernel(dst_hbm, w_hbm, z_hbm, out_hbm, dst_v, w_v, deg_sh, sem0, sem1):
    c = lax.axis_index("c")
    s = lax.axis_index("s")
    chunk = NPAD // NS
    pltpu.sync_copy(z_hbm.at[pl.ds(s * chunk, chunk)],
                    deg_sh.at[pl.ds(s * chunk, chunk)])

    wid = s * NC + c
    base = wid * ROWS_PW
    cp0 = pltpu.async_copy(dst_hbm.at[pl.ds(base, ROWS_PW)], dst_v, sem0)
    cp1 = pltpu.async_copy(w_hbm.at[pl.ds(base, ROWS_PW)], w_v, sem1)
    cp0.wait()
    cp1.wait()
    plsc.subcore_barrier()

    def body(r, _):
        pltpu.sync_copy(w_v.at[r], deg_sh.at[dst_v.at[r]], add=True)
        return 0

    lax.fori_loop(0, ROWS_PW, body, 0)
    plsc.subcore_barrier()
    pltpu.sync_copy(deg_sh.at[pl.ds(s * chunk, chunk)],
                    out_hbm.at[c, pl.ds(s * chunk, chunk)])


@functools.partial(
    pl.kernel,
    out_type=jax.ShapeDtypeStruct((NC, NPAD, H), jnp.float32),
    mesh=_sc_mesh,
    scratch_types=[
        pltpu.VMEM((SR, LANES), jnp.int32),
        pltpu.VMEM((SR, LANES), jnp.int32),
        pltpu.VMEM((SR, LANES), jnp.float32),
        pltpu.VMEM((2, LANES, H), jnp.float32),
        pltpu.VMEM((2, LANES, H), jnp.float32),
        pltpu.VMEM_SHARED((NPAD, H), jnp.float32),
        pltpu.VMEM_SHARED((N, H), jnp.float32),
        pltpu.SemaphoreType.DMA,
        pltpu.SemaphoreType.DMA,
        pltpu.SemaphoreType.DMA,
        pltpu.SemaphoreType.DMA,
    ],
    compiler_params=_sc_params,
)
def _edge_kernel(xs_hbm, src_hbm, dst_hbm, w_hbm, z_hbm, out_hbm,
                 src_v, dst_v, w_v, gbuf, sbuf, acc_sh, xs_sh,
                 g0, g1, s0, s1):
    c = lax.axis_index("c")
    s = lax.axis_index("s")
    nrows = NPAD // NS
    gsem = (g0, g1)
    ssem = (s0, s1)

    cpz = pltpu.async_copy(z_hbm.at[pl.ds(s * nrows, nrows)],
                           acc_sh.at[pl.ds(s * nrows, nrows)], s1)
    pltpu.sync_copy(xs_hbm.at[pl.ds(s * (N // NS), N // NS)],
                    xs_sh.at[pl.ds(s * (N // NS), N // NS)])
    cpz.wait()

    slow = c == SLOW_C
    nstages = jnp.where(slow, SLOW_STAGES, STAGES_PW - SLOW_STAGES)
    base = jnp.where(
        slow,
        s * SLOW_STAGES * SR,
        NS * SLOW_STAGES * SR + s * (STAGES_PW - SLOW_STAGES) * SR)
    plsc.subcore_barrier()

    def gather(r, b):
        pltpu.async_copy(xs_sh.at[src_v.at[r]], gbuf.at[b], gsem[b])

    def wait_gather(r, b):
        pltpu.make_async_copy(xs_sh.at[src_v.at[r]], gbuf.at[b],
                              gsem[b]).wait()

    def scale(r, b):
        def grp(g, _):
            wv = w_v[r, pl.ds(g * 16, 16)]
            for j in range(16):
                e = g * 16 + j
                we = wv[j]
                sbuf[b, e, pl.ds(0, 16)] = gbuf[b, e, pl.ds(0, 16)] * we
                sbuf[b, e, pl.ds(16, 16)] = gbuf[b, e, pl.ds(16, 16)] * we
                sbuf[b, e, pl.ds(32, 16)] = gbuf[b, e, pl.ds(32, 16)] * we
                sbuf[b, e, pl.ds(48, 16)] = gbuf[b, e, pl.ds(48, 16)] * we
            return 0

        lax.fori_loop(0, LANES // 16, grp, 0, unroll=2)

    def scatter(r, b):
        pltpu.async_copy(sbuf.at[b], acc_sh.at[dst_v.at[r]], ssem[b],
                         add=True)

    def wait_scatter(r, b):
        pltpu.make_async_copy(sbuf.at[b], acc_sh.at[dst_v.at[r]],
                              ssem[b]).wait()

    def stage(h, _):
        hb = base + h * SR
        cp0 = pltpu.async_copy(src_hbm.at[pl.ds(hb, SR)], src_v, g0)
        cp1 = pltpu.async_copy(dst_hbm.at[pl.ds(hb, SR)], dst_v, g1)
        cp2 = pltpu.async_copy(w_hbm.at[pl.ds(hb, SR)], w_v, s0)
        cp0.wait()
        cp1.wait()
        cp2.wait()

        for b in range(2):
            gather(b, b)

        def body(i, _):
            for b in range(2):
                r = 2 * i + b
                wait_gather(r, b)

                @pl.when(i > 0)
                def _():
                    wait_scatter(r - 2, b)

                scale(r, b)
                scatter(r, b)

                @pl.when(i < SR // 2 - 1)
                def _():
                    gather(r + 2, b)

            return 0

        lax.fori_loop(0, SR // 2, body, 0)
        for b in range(2):
            wait_scatter(SR - 2 + b, b)
        return 0

    lax.fori_loop(0, nstages, stage, 0)
    plsc.subcore_barrier()
    pltpu.sync_copy(acc_sh.at[pl.ds(s * nrows, nrows)],
                    out_hbm.at[c, pl.ds(s * nrows, nrows)])


_RB = 1000
_GRID = (N // _RB,)


def _tc1_body(deg0_ref, deg1_ref, x_ref, w1_ref, dinv_ref, xs_ref):
    deg = 1.0 + deg0_ref[...] + deg1_ref[...]
    dinv = jnp.where(deg > 0, lax.rsqrt(jnp.maximum(deg, 1e-12)), 0.0)
    dinv_ref[...] = dinv
    xw = jnp.dot(x_ref[...], w1_ref[...], preferred_element_type=jnp.float32)
    xs_ref[...] = xw * dinv


def _tc1(deg0, deg1, x, W1):
    return pl.pallas_call(
        _tc1_body,
        grid=_GRID,
        in_specs=[
            pl.BlockSpec((_RB, 1), lambda i: (i, 0)),
            pl.BlockSpec((_RB, 1), lambda i: (i, 0)),
            pl.BlockSpec((_RB, D), lambda i: (i, 0)),
            pl.BlockSpec((D, H), lambda i: (0, 0)),
        ],
        out_specs=[
            pl.BlockSpec((_RB, 1), lambda i: (i, 0)),
            pl.BlockSpec((_RB, H), lambda i: (i, 0)),
        ],
        out_shape=[
            jax.ShapeDtypeStruct((N, 1), jnp.float32),
            jax.ShapeDtypeStruct((N, H), jnp.float32),
        ],
    )(deg0, deg1, x, W1)


def _tc2_body(p_ref, q_ref, xs_ref, dinv_ref, b_ref, w_ref, out_ref):
    dinv = dinv_ref[...]
    h = dinv * (p_ref[0] + q_ref[0] + xs_ref[...]) + b_ref[...]
    h = jnp.maximum(h, 0.0)
    xw = jnp.dot(h, w_ref[...], preferred_element_type=jnp.float32)
    out_ref[...] = xw * dinv


def _tc2(p, xs, dinv, b1, W2):
    return pl.pallas_call(
        _tc2_body,
        grid=_GRID,
        in_specs=[
            pl.BlockSpec((1, _RB, H), lambda i: (0, i, 0)),
            pl.BlockSpec((1, _RB, H), lambda i: (1, i, 0)),
            pl.BlockSpec((_RB, H), lambda i: (i, 0)),
            pl.BlockSpec((_RB, 1), lambda i: (i, 0)),
            pl.BlockSpec((1, H), lambda i: (0, 0)),
            pl.BlockSpec((H, H), lambda i: (0, 0)),
        ],
        out_specs=pl.BlockSpec((_RB, H), lambda i: (i, 0)),
        out_shape=jax.ShapeDtypeStruct((N, H), jnp.float32),
    )(p, p, xs, dinv, b1, W2)


def _tc3_body(q0_ref, q1_ref, xs_ref, dinv_ref, b2_ref, wm1_ref, bm1_ref,
              wm2_ref, bm2_ref, out_ref):
    h = dinv_ref[...] * (q0_ref[0] + q1_ref[0] + xs_ref[...]) + b2_ref[...]
    h = jnp.maximum(h, 0.0)
    h = jnp.dot(h, wm1_ref[...], preferred_element_type=jnp.float32) + bm1_ref[...]
    h = jnp.maximum(h, 0.0)
    logits = jnp.dot(h, wm2_ref[...], preferred_element_type=jnp.float32) + bm2_ref[...]
    m = jnp.max(logits, axis=-1, keepdims=True)
    e = jnp.exp(logits - m)
    out_ref[...] = e / jnp.sum(e, axis=-1, keepdims=True)


def _tc3(q0, q1, xs, dinv, b2, Wm1, bm1, Wm2, bm2):
    return pl.pallas_call(
        _tc3_body,
        grid=_GRID,
        in_specs=[
            pl.BlockSpec((1, _RB, H), lambda i: (0, i, 0)),
            pl.BlockSpec((1, _RB, H), lambda i: (1, i, 0)),
            pl.BlockSpec((_RB, H), lambda i: (i, 0)),
            pl.BlockSpec((_RB, 1), lambda i: (i, 0)),
            pl.BlockSpec((1, H), lambda i: (0, 0)),
            pl.BlockSpec((H, H), lambda i: (0, 0)),
            pl.BlockSpec((1, H), lambda i: (0, 0)),
            pl.BlockSpec((H, C), lambda i: (0, 0)),
            pl.BlockSpec((1, C), lambda i: (0, 0)),
        ],
        out_specs=pl.BlockSpec((_RB, C), lambda i: (i, 0)),
        out_shape=jax.ShapeDtypeStruct((N, C), jnp.float32),
    )(q0, q1, xs, dinv, b2, Wm1, bm1, Wm2, bm2)


def kernel(x, edge_index, edge_attr, W1, b1, W2, b2, Wm1, bm1, Wm2, bm2):
    pad = EPAD - E
    src = jnp.concatenate(
        [edge_index[0], jnp.zeros((pad,), jnp.int32)]).reshape(ROWS_PAD, LANES)
    dst = jnp.concatenate(
        [edge_index[1], jnp.zeros((pad,), jnp.int32)]).reshape(ROWS_PAD, LANES)
    w = jnp.concatenate(
        [edge_attr, jnp.zeros((pad,), jnp.float32)]).reshape(ROWS_PAD, LANES)

    zcol = jnp.zeros((NPAD,), jnp.float32)
    zacc = jnp.zeros((NPAD, H), jnp.float32)
    degp = _deg_kernel(dst, w, zcol)
    deg0 = degp[0, :N].reshape(N, 1)
    deg1 = degp[1, :N].reshape(N, 1)

    dinv, xs1 = _tc1(deg0, deg1, x, W1)

    p = _edge_kernel(xs1, src, dst, w, zacc)
    xs2 = _tc2(p, xs1, dinv, b1.reshape(1, H), W2)

    q = _edge_kernel(xs2, src, dst, w, zacc)
    return _tc3(q, q, xs2, dinv, b2.reshape(1, H),
                Wm1, bm1.reshape(1, H), Wm2, bm2.reshape(1, C))
